# Initial kernel scaffold; baseline (speedup 1.0000x reference)
import jax
import jax.numpy as jnp
from jax.experimental import pallas as pl


def kernel(u_emb, v_emb, pos_u, pos_v, neg_v):
    raise NotImplementedError("write your pallas kernel here")


# speedup vs baseline: 5.3279x; 5.3279x over previous
import functools

import jax
import jax.numpy as jnp
from jax import lax
from jax.experimental import pallas as pl
from jax.experimental.pallas import tpu as pltpu
from jax.experimental.pallas import tpu_sc as plsc

B = 16384
NEG = 20
D = 64
NC = 2
NS = 16
NW = NC * NS
BPW = B // NW
CHUNK = 64
NCHUNK = BPW // CHUNK
L = 16
NJ = D // L


def _sc_partials(u_emb, v_emb, pos_u, pos_v, neg_v_flat):
    mesh = plsc.VectorSubcoreMesh(core_axis_name="c", subcore_axis_name="s")

    @functools.partial(
        pl.kernel,
        out_type=(
            jax.ShapeDtypeStruct((B * L,), jnp.float32),
            jax.ShapeDtypeStruct((B * L,), jnp.float32),
        ),
        mesh=mesh,
        compiler_params=pltpu.CompilerParams(use_tc_tiling_on_sc=False),
        scratch_types=[
            pltpu.VMEM((CHUNK,), jnp.int32),
            pltpu.VMEM((CHUNK,), jnp.int32),
            pltpu.VMEM((CHUNK * NEG,), jnp.int32),
            pltpu.VMEM((CHUNK, D), jnp.float32),
            pltpu.VMEM((CHUNK, D), jnp.float32),
            pltpu.VMEM((CHUNK * NEG, D), jnp.float32),
            pltpu.VMEM((BPW * L,), jnp.float32),
            pltpu.VMEM((BPW * L,), jnp.float32),
            pltpu.SemaphoreType.DMA,
            pltpu.SemaphoreType.DMA,
            pltpu.SemaphoreType.DMA,
        ],
    )
    def k(u_hbm, v_hbm, pu_hbm, pv_hbm, nv_hbm, pos_out, neg_out,
          idxu, idxv, idxn, urows, vrows, nrows, psc, nsc,
          sem_u, sem_v, sem_n):
        wid = lax.axis_index("s") * NC + lax.axis_index("c")
        base = wid * BPW

        def chunk_body(ci, _):
            cb = base + ci * CHUNK
            pltpu.sync_copy(pu_hbm.at[pl.ds(cb, CHUNK)], idxu)
            pltpu.sync_copy(pv_hbm.at[pl.ds(cb, CHUNK)], idxv)
            pltpu.sync_copy(nv_hbm.at[pl.ds(cb * NEG, CHUNK * NEG)], idxn)
            cp_u = pltpu.async_copy(u_hbm.at[idxu], urows, sem_u)
            cp_v = pltpu.async_copy(v_hbm.at[idxv], vrows, sem_v)
            cp_n = pltpu.async_copy(v_hbm.at[idxn], nrows, sem_n)
            cp_u.wait()
            cp_v.wait()
            cp_n.wait()

            def elem_body(e, _):
                u = [urows[e, pl.ds(j * L, L)] for j in range(NJ)]
                v = [vrows[e, pl.ds(j * L, L)] for j in range(NJ)]
                p = u[0] * v[0]
                for j in range(1, NJ):
                    p = p + u[j] * v[j]
                ne = e * NEG
                acc = [nrows[ne, pl.ds(j * L, L)] for j in range(NJ)]
                for kk in range(1, NEG):
                    for j in range(NJ):
                        acc[j] = acc[j] + nrows[ne + kk, pl.ds(j * L, L)]
                sn = acc[0] * u[0]
                for j in range(1, NJ):
                    sn = sn + acc[j] * u[j]
                off = (ci * CHUNK + e) * L
                psc[pl.ds(off, L)] = p
                nsc[pl.ds(off, L)] = sn
                return 0

            lax.fori_loop(0, CHUNK, elem_body, 0)
            return 0

        lax.fori_loop(0, NCHUNK, chunk_body, 0)
        pltpu.sync_copy(psc, pos_out.at[pl.ds(base * L, BPW * L)])
        pltpu.sync_copy(nsc, neg_out.at[pl.ds(base * L, BPW * L)])

    return k(u_emb, v_emb, pos_u, pos_v, neg_v_flat)


def _loss_body(pos_ref, neg_ref, out_ref):
    rows = lax.broadcasted_iota(jnp.int32, (128, 8), 0)
    cols = lax.broadcasted_iota(jnp.int32, (128, 8), 1)
    m = ((rows // L) == cols).astype(jnp.float32)
    sp = jnp.dot(pos_ref[...], m, preferred_element_type=jnp.float32)
    sn = -jnp.dot(neg_ref[...], m, preferred_element_type=jnp.float32)
    lsp = jnp.minimum(sp, 0.0) - jnp.log1p(jnp.exp(-jnp.abs(sp)))
    lsn = jnp.minimum(sn, 0.0) - jnp.log1p(jnp.exp(-jnp.abs(sn)))
    out_ref[0, 0] = -(jnp.sum(lsp) + jnp.sum(lsn))


def _tc_loss(pos_part, neg_part):
    p2 = pos_part.reshape(B * L // 128, 128)
    n2 = neg_part.reshape(B * L // 128, 128)
    out = pl.pallas_call(
        _loss_body,
        out_shape=jax.ShapeDtypeStruct((1, 1), jnp.float32),
        out_specs=pl.BlockSpec(memory_space=pltpu.SMEM),
    )(p2, n2)
    return out[0, 0]


@jax.jit
def kernel(u_emb, v_emb, pos_u, pos_v, neg_v):
    pos_part, neg_part = _sc_partials(
        u_emb, v_emb, pos_u, pos_v, neg_v.reshape(-1))
    return _tc_loss(pos_part, neg_part)
